# baseline (speedup 1.0000x reference)
import functools

import jax
import jax.numpy as jnp
from jax import lax
from jax.experimental import pallas as pl

N_ROWS = 16384
N_CLS = 1000
RATE_KEEP = 13107
BLOCK_ROWS = 256


def _ce_block(x_ref, t_ref, o_ref):
    x = x_ref[:]
    t = t_ref[:]
    m = jnp.max(x, axis=1, keepdims=True)
    s = jnp.sum(jnp.exp(x - m), axis=1, keepdims=True)
    lse = m + jnp.log(s)
    cls_ids = lax.broadcasted_iota(jnp.int32, x.shape, 1)
    tv = jnp.sum(jnp.where(cls_ids == t, x, 0.0), axis=1, keepdims=True)
    o_ref[:] = lse - tv


def _select_topk(p_ref, o_ref, *, k):
    v = p_ref[:]
    bits = lax.bitcast_convert_type(v, jnp.int32)

    def body(_, carry):
        lo, hi = carry
        mid = lo + (hi - lo) // 2
        cnt = jnp.sum((bits >= mid).astype(jnp.int32))
        take = cnt >= k
        return jnp.where(take, mid, lo), jnp.where(take, hi, mid)

    lo, _ = lax.fori_loop(0, 31, body, (jnp.int32(0), jnp.int32(0x7F800000)))
    t = lax.bitcast_convert_type(lo, jnp.float32)
    gt = bits > lo
    cnt_gt = jnp.sum(gt.astype(jnp.int32))
    sum_gt = jnp.sum(jnp.where(gt, v, 0.0))
    loss = (sum_gt + (k - cnt_gt).astype(jnp.float32) * t) / k
    o_ref[:] = jnp.reshape(loss, (1, 1))


@jax.jit
def kernel(cls_pred, cls_target):
    tgt = cls_target.astype(jnp.int32).reshape(N_ROWS, 1)
    per = pl.pallas_call(
        _ce_block,
        grid=(N_ROWS // BLOCK_ROWS,),
        in_specs=[
            pl.BlockSpec((BLOCK_ROWS, N_CLS), lambda i: (i, 0)),
            pl.BlockSpec((BLOCK_ROWS, 1), lambda i: (i, 0)),
        ],
        out_specs=pl.BlockSpec((BLOCK_ROWS, 1), lambda i: (i, 0)),
        out_shape=jax.ShapeDtypeStruct((N_ROWS, 1), jnp.float32),
    )(cls_pred, tgt)

    loss = pl.pallas_call(
        functools.partial(_select_topk, k=RATE_KEEP),
        out_shape=jax.ShapeDtypeStruct((1, 1), jnp.float32),
    )(per.reshape(128, 128))
    return loss[0, 0]


# speedup vs baseline: 1.2989x; 1.2989x over previous
import functools

import jax
import jax.numpy as jnp
from jax import lax
from jax.experimental import pallas as pl

N_ROWS = 16384
N_CLS = 1000
RATE_KEEP = 13107
BLOCK_ROWS = 2048


def _ce_block(x_ref, t_ref, o_ref):
    x = x_ref[:]
    t = t_ref[:]
    m = jnp.max(x, axis=1, keepdims=True)
    s = jnp.sum(jnp.exp(x - m), axis=1, keepdims=True)
    lse = m + jnp.log(s)
    cls_ids = lax.broadcasted_iota(jnp.int32, x.shape, 1)
    tv = jnp.sum(jnp.where(cls_ids == t, x, 0.0), axis=1, keepdims=True)
    o_ref[:] = lse - tv


def _select_topk(p_ref, o_ref, *, k):
    v = p_ref[:]
    bits = lax.bitcast_convert_type(v, jnp.int32)

    def body(_, carry):
        lo, hi = carry
        mid = lo + (hi - lo) // 2
        cnt = jnp.sum((bits >= mid).astype(jnp.int32))
        take = cnt >= k
        return jnp.where(take, mid, lo), jnp.where(take, hi, mid)

    lo, _ = lax.fori_loop(0, 31, body, (jnp.int32(0), jnp.int32(0x7F800000)))
    t = lax.bitcast_convert_type(lo, jnp.float32)
    gt = bits > lo
    cnt_gt = jnp.sum(gt.astype(jnp.int32))
    sum_gt = jnp.sum(jnp.where(gt, v, 0.0))
    loss = (sum_gt + (k - cnt_gt).astype(jnp.float32) * t) / k
    o_ref[:] = jnp.reshape(loss, (1, 1))


@jax.jit
def kernel(cls_pred, cls_target):
    tgt = cls_target.astype(jnp.int32).reshape(N_ROWS, 1)
    per = pl.pallas_call(
        _ce_block,
        grid=(N_ROWS // BLOCK_ROWS,),
        in_specs=[
            pl.BlockSpec((BLOCK_ROWS, N_CLS), lambda i: (i, 0)),
            pl.BlockSpec((BLOCK_ROWS, 1), lambda i: (i, 0)),
        ],
        out_specs=pl.BlockSpec((BLOCK_ROWS, 1), lambda i: (i, 0)),
        out_shape=jax.ShapeDtypeStruct((N_ROWS, 1), jnp.float32),
    )(cls_pred, tgt)

    loss = pl.pallas_call(
        functools.partial(_select_topk, k=RATE_KEEP),
        out_shape=jax.ShapeDtypeStruct((1, 1), jnp.float32),
    )(per.reshape(128, 128))
    return loss[0, 0]


# speedup vs baseline: 1.4534x; 1.1189x over previous
import jax
import jax.numpy as jnp
from jax import lax
from jax.experimental import pallas as pl
from jax.experimental.pallas import tpu as pltpu

N_ROWS = 16384
N_CLS = 1000
RATE_KEEP = 13107
BLOCK_ROWS = 2048
N_BLOCKS = N_ROWS // BLOCK_ROWS
SUB_ROWS = BLOCK_ROWS // 128


def _ohem_block(x_ref, t_ref, o_ref, acc_ref):
    x = x_ref[:]
    t = t_ref[:]
    m = jnp.max(x, axis=1, keepdims=True)
    s = jnp.sum(jnp.exp(x - m), axis=1, keepdims=True)
    lse = m + jnp.log(s)
    cls_ids = lax.broadcasted_iota(jnp.int32, x.shape, 1)
    tv = jnp.sum(jnp.where(cls_ids == t, x, 0.0), axis=1, keepdims=True)
    per = lse - tv

    i = pl.program_id(0)
    acc_ref[pl.ds(i * SUB_ROWS, SUB_ROWS), :] = jnp.reshape(per, (SUB_ROWS, 128))

    @pl.when(i == N_BLOCKS - 1)
    def _select():
        k = RATE_KEEP
        v = acc_ref[:]
        bits = lax.bitcast_convert_type(v, jnp.int32)

        def body(_, carry):
            lo, hi = carry
            mid = lo + (hi - lo) // 2
            cnt = jnp.sum((bits >= mid).astype(jnp.int32))
            take = cnt >= k
            return jnp.where(take, mid, lo), jnp.where(take, hi, mid)

        lo, _ = lax.fori_loop(0, 31, body, (jnp.int32(0), jnp.int32(0x7F800000)))
        thr = lax.bitcast_convert_type(lo, jnp.float32)
        gt = bits > lo
        cnt_gt = jnp.sum(gt.astype(jnp.int32))
        sum_gt = jnp.sum(jnp.where(gt, v, 0.0))
        loss = (sum_gt + (k - cnt_gt).astype(jnp.float32) * thr) / k
        o_ref[:] = jnp.reshape(loss, (1, 1))


@jax.jit
def kernel(cls_pred, cls_target):
    tgt = cls_target.astype(jnp.int32).reshape(N_ROWS, 1)
    loss = pl.pallas_call(
        _ohem_block,
        grid=(N_BLOCKS,),
        in_specs=[
            pl.BlockSpec((BLOCK_ROWS, N_CLS), lambda i: (i, 0)),
            pl.BlockSpec((BLOCK_ROWS, 1), lambda i: (i, 0)),
        ],
        out_specs=pl.BlockSpec((1, 1), lambda i: (0, 0)),
        out_shape=jax.ShapeDtypeStruct((1, 1), jnp.float32),
        scratch_shapes=[pltpu.VMEM((128, 128), jnp.float32)],
    )(cls_pred, tgt)
    return loss[0, 0]
